# baseline (speedup 1.0000x reference)
import math

import jax
import jax.numpy as jnp
from jax import lax
from jax.experimental import pallas as pl
from jax.experimental.pallas import tpu as pltpu
from jax.experimental.pallas import tpu_sc as plsc

N = 10000
E = 160000
HID = 256
EDGE = 128
H = 8
D = HID // H

NC = 2
NS = 16
NW = NC * NS

F32 = jnp.float32


def _ln(x, g, b):
    m = jnp.mean(x, axis=-1, keepdims=True)
    xm = x - m
    v = jnp.mean(xm * xm, axis=-1, keepdims=True)
    return xm * lax.rsqrt(v + 1e-5) * g + b


BN = 400


def _node_body(x_ref, g_ref, b_ref, wq_ref, wk_ref, wv_ref, q_ref, k_ref, v_ref):
    h = _ln(x_ref[...], g_ref[...], b_ref[...])
    q_ref[...] = jnp.dot(h, wq_ref[...], preferred_element_type=F32)
    k_ref[...] = jnp.dot(h, wk_ref[...], preferred_element_type=F32)
    v_ref[...] = jnp.dot(h, wv_ref[...], preferred_element_type=F32)


def _tc_nodes(x, n1_g, n1_b, WqT, WkT, WvT):
    row = pl.BlockSpec((BN, HID), lambda i: (i, 0))
    full = pl.BlockSpec((HID, HID), lambda i: (0, 0))
    vec = pl.BlockSpec((1, HID), lambda i: (0, 0))
    return pl.pallas_call(
        _node_body,
        grid=(N // BN,),
        in_specs=[row, vec, vec, full, full, full],
        out_specs=[row, row, row],
        out_shape=[jax.ShapeDtypeStruct((N, HID), F32)] * 3,
    )(x, n1_g.reshape(1, HID), n1_b.reshape(1, HID), WqT, WkT, WvT)


EPW = E // NW
GB = 200
NCH = EPW // GB


def _gather_table(tab, out, idx_hbm, base0, iA, iB, rA, rB, sA, sB):
    pltpu.sync_copy(idx_hbm.at[pl.ds(base0, GB)], iA)
    pltpu.async_copy(tab.at[iA], rA, sA)
    pltpu.sync_copy(idx_hbm.at[pl.ds(base0 + GB, GB)], iB)
    pltpu.async_copy(tab.at[iB], rB, sB)

    def body(j, carry):
        cA = base0 + (2 * j) * GB
        pltpu.make_async_copy(tab.at[iA], rA, sA).wait()
        pltpu.sync_copy(rA, out.at[pl.ds(cA, GB)])

        @pl.when(j < NCH // 2)
        def _():
            pltpu.sync_copy(idx_hbm.at[pl.ds(cA + 2 * GB, GB)], iA)
            pltpu.async_copy(tab.at[iA], rA, sA)
            pltpu.make_async_copy(tab.at[iB], rB, sB).wait()
            pltpu.sync_copy(rB, out.at[pl.ds(cA + GB, GB)])

        @pl.when(j < NCH // 2 - 1)
        def _():
            pltpu.sync_copy(idx_hbm.at[pl.ds(cA + 3 * GB, GB)], iB)
            pltpu.async_copy(tab.at[iB], rB, sB)

        return carry

    lax.fori_loop(0, NCH // 2 + 1, body, 0)


def _sc_gather_body(q_hbm, k_hbm, v_hbm, dst_hbm, src_hbm,
                    qg_hbm, kg_hbm, vg_hbm, iA, iB, rA, rB, sA, sB):
    c = lax.axis_index("c")
    s = lax.axis_index("s")
    base0 = (s * NC + c) * EPW
    _gather_table(q_hbm, qg_hbm, dst_hbm, base0, iA, iB, rA, rB, sA, sB)
    _gather_table(k_hbm, kg_hbm, src_hbm, base0, iA, iB, rA, rB, sA, sB)
    _gather_table(v_hbm, vg_hbm, src_hbm, base0, iA, iB, rA, rB, sA, sB)


def _sc_gather(Q, K, V, dst, src):
    f = pl.kernel(
        _sc_gather_body,
        out_type=[jax.ShapeDtypeStruct((E, HID), F32)] * 3,
        mesh=plsc.VectorSubcoreMesh(core_axis_name="c", subcore_axis_name="s"),
        scratch_types=[
            pltpu.VMEM((GB,), jnp.int32),
            pltpu.VMEM((GB,), jnp.int32),
            pltpu.VMEM((GB, HID), F32),
            pltpu.VMEM((GB, HID), F32),
            pltpu.SemaphoreType.DMA,
            pltpu.SemaphoreType.DMA,
        ],
    )
    return f(Q, K, V, dst, src)


BE = 1600


def _ew_body(ea_ref, w1_ref, b1_ref, w2_ref, b2_ref, ew_ref):
    t = jnp.dot(ea_ref[...], w1_ref[...], preferred_element_type=F32) + b1_ref[...]
    t = t * jax.nn.sigmoid(t)
    ew_ref[...] = jnp.dot(t, w2_ref[...], preferred_element_type=F32) + b2_ref[...]


def _tc_ew(edge_attr, w1T, b1, w2T, b2):
    erow = pl.BlockSpec((BE, EDGE), lambda i: (i, 0))
    hrow = pl.BlockSpec((BE, HID), lambda i: (i, 0))
    w1s = pl.BlockSpec((EDGE, HID), lambda i: (0, 0))
    w2s = pl.BlockSpec((HID, HID), lambda i: (0, 0))
    vec = pl.BlockSpec((1, HID), lambda i: (0, 0))
    return pl.pallas_call(
        _ew_body,
        grid=(E // BE,),
        in_specs=[erow, w1s, vec, w2s, vec],
        out_specs=hrow,
        out_shape=jax.ShapeDtypeStruct((E, HID), F32),
    )(edge_attr, w1T, b1.reshape(1, HID), w2T, b2.reshape(1, HID))


def _msg_body(qg_ref, kg_ref, vg_ref, ew_ref, msg_ref):
    ew = ew_ref[...]
    s = qg_ref[...] * kg_ref[...] * ew
    lane = lax.broadcasted_iota(jnp.int32, (HID, H), 0)
    head = lax.broadcasted_iota(jnp.int32, (HID, H), 1)
    G = (lane // D == head).astype(F32)
    hs = jnp.dot(s, G, preferred_element_type=F32) * (1.0 / math.sqrt(D))
    attn = jax.nn.sigmoid(hs)
    alane = jnp.dot(attn, G.T, preferred_element_type=F32)
    msg_ref[...] = alane * vg_ref[...] * ew


def _tc_msg(qg, kg, vg, ew):
    hrow = pl.BlockSpec((BE, HID), lambda i: (i, 0))
    return pl.pallas_call(
        _msg_body,
        grid=(E // BE,),
        in_specs=[hrow, hrow, hrow, hrow],
        out_specs=hrow,
        out_shape=jax.ShapeDtypeStruct((E, HID), F32),
    )(qg, kg, vg, ew)


EPS = E // NS
SB = 40
SCH = EPS // SB
NPAD = 10240
NPS = NPAD // NS
CW = HID // NC


def _sc_scatter_body(msg_hbm, dst_hbm, zeros_hbm, agg_hbm,
                     iA, iB, bA, bB, shared, sA, sB):
    c = lax.axis_index("c")
    s = lax.axis_index("s")
    pltpu.sync_copy(zeros_hbm.at[pl.ds(s * NPS, NPS)], shared.at[pl.ds(s * NPS, NPS)])
    plsc.subcore_barrier()
    base0 = s * EPS
    col = c * CW

    pltpu.sync_copy(dst_hbm.at[pl.ds(base0, SB)], iA)
    pltpu.async_copy(msg_hbm.at[pl.ds(base0, SB), pl.ds(col, CW)], bA, sA)
    pltpu.sync_copy(dst_hbm.at[pl.ds(base0 + SB, SB)], iB)
    pltpu.async_copy(msg_hbm.at[pl.ds(base0 + SB, SB), pl.ds(col, CW)], bB, sB)

    def body(j, carry):
        cA = base0 + (2 * j) * SB
        pltpu.make_async_copy(msg_hbm.at[pl.ds(cA, SB), pl.ds(col, CW)], bA, sA).wait()
        pltpu.sync_copy(bA, shared.at[iA], add=True)

        @pl.when(j < SCH // 2 - 1)
        def _():
            pltpu.sync_copy(dst_hbm.at[pl.ds(cA + 2 * SB, SB)], iA)
            pltpu.async_copy(msg_hbm.at[pl.ds(cA + 2 * SB, SB), pl.ds(col, CW)], bA, sA)

        pltpu.make_async_copy(msg_hbm.at[pl.ds(cA + SB, SB), pl.ds(col, CW)], bB, sB).wait()
        pltpu.sync_copy(bB, shared.at[iB], add=True)

        @pl.when(j < SCH // 2 - 1)
        def _():
            pltpu.sync_copy(dst_hbm.at[pl.ds(cA + 3 * SB, SB)], iB)
            pltpu.async_copy(msg_hbm.at[pl.ds(cA + 3 * SB, SB), pl.ds(col, CW)], bB, sB)

        return carry

    lax.fori_loop(0, SCH // 2, body, 0)
    plsc.subcore_barrier()
    pltpu.sync_copy(shared.at[pl.ds(s * NPS, NPS)],
                    agg_hbm.at[pl.ds(s * NPS, NPS), pl.ds(col, CW)])


def _sc_scatter(msg, dst):
    zeros = jnp.zeros((NPAD, CW), F32)
    f = pl.kernel(
        _sc_scatter_body,
        out_type=jax.ShapeDtypeStruct((NPAD, HID), F32),
        mesh=plsc.VectorSubcoreMesh(core_axis_name="c", subcore_axis_name="s"),
        scratch_types=[
            pltpu.VMEM((SB,), jnp.int32),
            pltpu.VMEM((SB,), jnp.int32),
            pltpu.VMEM((SB, CW), F32),
            pltpu.VMEM((SB, CW), F32),
            pltpu.VMEM_SHARED((NPAD, CW), F32),
            pltpu.SemaphoreType.DMA,
            pltpu.SemaphoreType.DMA,
        ],
    )
    return f(msg, dst, zeros)[:N]


def _final_body(x_ref, agg_ref, ow_ref, ob_ref, g2_ref, b2_ref,
                fw1_ref, fb1_ref, fw2_ref, fb2_ref, out_ref):
    y = x_ref[...] + jnp.dot(agg_ref[...], ow_ref[...], preferred_element_type=F32) + ob_ref[...]
    h2 = _ln(y, g2_ref[...], b2_ref[...])
    ff = jnp.dot(h2, fw1_ref[...], preferred_element_type=F32) + fb1_ref[...]
    ff = 0.5 * ff * (1.0 + lax.erf(ff * (1.0 / math.sqrt(2.0))))
    ff = jnp.dot(ff, fw2_ref[...], preferred_element_type=F32) + fb2_ref[...]
    out_ref[...] = y + ff


def _tc_final(x, agg, out_wT, out_b, n2_g, n2_b, ff_w1T, ff_b1, ff_w2T, ff_b2):
    row = pl.BlockSpec((BN, HID), lambda i: (i, 0))
    full = pl.BlockSpec((HID, HID), lambda i: (0, 0))
    vec = pl.BlockSpec((1, HID), lambda i: (0, 0))
    w1s = pl.BlockSpec((HID, 4 * HID), lambda i: (0, 0))
    v1s = pl.BlockSpec((1, 4 * HID), lambda i: (0, 0))
    w2s = pl.BlockSpec((4 * HID, HID), lambda i: (0, 0))
    return pl.pallas_call(
        _final_body,
        grid=(N // BN,),
        in_specs=[row, row, full, vec, vec, vec, w1s, v1s, w2s, vec],
        out_specs=row,
        out_shape=jax.ShapeDtypeStruct((N, HID), F32),
    )(x, agg, out_wT, out_b.reshape(1, HID), n2_g.reshape(1, HID),
      n2_b.reshape(1, HID), ff_w1T, ff_b1.reshape(1, 4 * HID), ff_w2T,
      ff_b2.reshape(1, HID))


def kernel(x, edge_index, edge_attr, Wq, Wk, Wv, ep_w1, ep_b1, ep_w2, ep_b2,
           out_w, out_b, n1_g, n1_b, n2_g, n2_b, ff_w1, ff_b1, ff_w2, ff_b2):
    src = edge_index[0]
    dst = edge_index[1]
    ew = _tc_ew(edge_attr, ep_w1.T, ep_b1, ep_w2.T, ep_b2)
    Q, K, V = _tc_nodes(x, n1_g, n1_b, Wq.T, Wk.T, Wv.T)
    qg, kg, vg = _sc_gather(Q, K, V, dst, src)
    msg = _tc_msg(qg, kg, vg, ew)
    agg = _sc_scatter(msg, dst)
    return _tc_final(x, agg, out_w.T, out_b, n2_g, n2_b,
                     ff_w1.T, ff_b1, ff_w2.T, ff_b2)
